# baseline (speedup 1.0000x reference)
import jax
import jax.numpy as jnp
from jax import lax
from jax.experimental import pallas as pl
from jax.experimental.pallas import tpu as pltpu
from jax.experimental.pallas import tpu_sc as plsc

_N = 262144
_L = 128
_R = _N // _L
_NC = 2
_NS = 16
_NW = _NC * _NS
_EPT = _N // _NW
_CH = 128
_NCHUNK = _EPT // _CH
_HI_BITS = 0x43000000
_LN2 = 0.6931471805599453


def _softlog(x):
    bits = lax.bitcast_convert_type(x, jnp.int32)
    e = ((bits >> 23) - 127).astype(jnp.float32)
    m = lax.bitcast_convert_type(
        (bits & 0x7FFFFF) | 0x3F800000, jnp.float32)
    r = (m - 1.0) / (m + 1.0)
    r2 = r * r
    lnm = 2.0 * r * (1.0 + r2 * (1.0 / 3.0 + r2 * (0.2 + r2 * (1.0 / 7.0))))
    return e * _LN2 + lnm


def _sc_body(pl_hbm, off_hbm, plm_hbm, gl_hbm, gb_hbm, glm_hbm,
             negv_out, part_out,
             lbl_b, p_b, off_b, gb_b, plm_b, glm_b, negc_b, part_b, sem):
    wid = lax.axis_index("s") * _NC + lax.axis_index("c")
    iota = lax.iota(jnp.int32, 16)
    zero16 = jnp.zeros((16,), jnp.int32)
    zf = jnp.zeros((16,), jnp.float32)

    np_a, nn_a, sp_a, sna_a, nb_a, nl_a = zf, zf, zf, zf, zf, zf
    bsq_a, lsq_a = zf, zf

    def chunk_body(c, carry):
        np_a, nn_a, sp_a, sna_a, nb_a, nl_a, bsq_a, lsq_a = carry
        s0 = wid * _EPT + c * _CH
        cps = [
            pltpu.make_async_copy(gl_hbm.at[pl.ds(s0, _CH)], lbl_b, sem),
            pltpu.make_async_copy(pl_hbm.at[pl.ds(s0, _CH), :], p_b, sem),
            pltpu.make_async_copy(off_hbm.at[pl.ds(s0, _CH), :], off_b, sem),
            pltpu.make_async_copy(gb_hbm.at[pl.ds(s0, _CH), :], gb_b, sem),
            pltpu.make_async_copy(plm_hbm.at[pl.ds(s0, _CH), :], plm_b, sem),
            pltpu.make_async_copy(glm_hbm.at[pl.ds(s0, _CH), :], glm_b, sem),
        ]
        for cp in cps:
            cp.start()
        for cp in cps:
            cp.wait()

        def bce_step(g, carry):
            np_a, nn_a, sp_a, sna_a, nb_a, nl_a = carry
            li = g * 16 + iota
            lbl = lbl_b[pl.ds(g * 16, 16)]
            p = plsc.load_gather(p_b, [li, zero16])
            lp = _softlog(p)
            l1p = _softlog(1.0 - p)
            loss_pos = -jnp.maximum(lp, -100.0)
            loss_neg = -jnp.maximum(l1p, -100.0)
            pos = lbl == 1
            neg = lbl == 0
            np_a = np_a + jnp.where(pos, 1.0, 0.0)
            nn_a = nn_a + jnp.where(neg, 1.0, 0.0)
            sp_a = sp_a + jnp.where(pos, loss_pos, 0.0)
            sna_a = sna_a + jnp.where(neg, loss_neg, 0.0)
            nb_a = nb_a + jnp.where(pos | (lbl == 2), 1.0, 0.0)
            nl_a = nl_a + jnp.where(lbl == -1, 1.0, 0.0)
            negc_b[pl.ds(g * 16, 16)] = jnp.where(neg, loss_neg, -1.0)
            return np_a, nn_a, sp_a, sna_a, nb_a, nl_a

        np_a, nn_a, sp_a, sna_a, nb_a, nl_a = lax.fori_loop(
            0, _CH // 16, bce_step,
            (np_a, nn_a, sp_a, sna_a, nb_a, nl_a), unroll=2)

        pltpu.sync_copy(negc_b, negv_out.at[pl.ds(s0, _CH)])

        def box_step(g, acc):
            q = g * 16 + iota
            ei = q >> 2
            co = q & 3
            a = plsc.load_gather(off_b, [ei, co])
            b = plsc.load_gather(gb_b, [ei, co])
            lbl = plsc.load_gather(lbl_b, [ei])
            d = a - b
            m = (lbl == 1) | (lbl == 2)
            return acc + jnp.where(m, d * d, 0.0)

        bsq_a = lax.fori_loop(0, _CH * 4 // 16, box_step, bsq_a, unroll=4)

        def land_step(g, acc):
            q = g * 16 + iota
            ei = q // 10
            co = q % 10
            a = plsc.load_gather(plm_b, [ei, co])
            b = plsc.load_gather(glm_b, [ei, co])
            lbl = plsc.load_gather(lbl_b, [ei])
            d = a - b
            return acc + jnp.where(lbl == -1, d * d, 0.0)

        lsq_a = lax.fori_loop(0, _CH * 10 // 16, land_step, lsq_a, unroll=4)
        return np_a, nn_a, sp_a, sna_a, nb_a, nl_a, bsq_a, lsq_a

    np_a, nn_a, sp_a, sna_a, nb_a, nl_a, bsq_a, lsq_a = lax.fori_loop(
        0, _NCHUNK, chunk_body,
        (np_a, nn_a, sp_a, sna_a, nb_a, nl_a, bsq_a, lsq_a))

    for j, vec in enumerate(
            (np_a, nn_a, sp_a, sna_a, nb_a, bsq_a, nl_a, lsq_a)):
        part_b[pl.ds(j * 16, 16)] = vec
    pltpu.sync_copy(part_b, part_out.at[pl.ds(wid * 128, 128)])


def _sc_stage(pred_label, pred_offset, pred_landmarks, gt_label, gt_boxes,
              gt_landmarks):
    return pl.kernel(
        _sc_body,
        out_type=(
            jax.ShapeDtypeStruct((_N,), jnp.float32),
            jax.ShapeDtypeStruct((_NW * 128,), jnp.float32),
        ),
        mesh=plsc.VectorSubcoreMesh(core_axis_name="c", subcore_axis_name="s"),
        compiler_params=pltpu.CompilerParams(needs_layout_passes=False, use_tc_tiling_on_sc=True),
        scratch_types=[
            pltpu.VMEM((_CH,), jnp.int32),
            pltpu.VMEM((_CH, 1), jnp.float32),
            pltpu.VMEM((_CH, 4), jnp.float32),
            pltpu.VMEM((_CH, 4), jnp.float32),
            pltpu.VMEM((_CH, 10), jnp.float32),
            pltpu.VMEM((_CH, 10), jnp.float32),
            pltpu.VMEM((_CH,), jnp.float32),
            pltpu.VMEM((128,), jnp.float32),
            pltpu.SemaphoreType.DMA,
        ],
    )(pred_label, pred_offset, pred_landmarks, gt_label, gt_boxes,
      gt_landmarks)


def _tc_body(negv_ref, part_ref, out_ref):
    part = part_ref[...]
    n_pos = jnp.sum(part[:, 0:16])
    n_neg = jnp.sum(part[:, 16:32])
    sum_pos = jnp.sum(part[:, 32:48])
    sum_neg_all = jnp.sum(part[:, 48:64])
    n_box = jnp.sum(part[:, 64:80])
    box_sq = jnp.sum(part[:, 80:96])
    n_land = jnp.sum(part[:, 96:112])
    land_sq = jnp.sum(part[:, 112:128])

    negv = negv_ref[...]
    bits = lax.bitcast_convert_type(negv, jnp.int32)
    k_i = n_pos.astype(jnp.int32)

    def step(_, carry):
        lo, hi = carry
        mid = (lo + hi) // 2
        cnt = jnp.sum((bits >= mid).astype(jnp.int32))
        ok = cnt >= k_i
        return jnp.where(ok, mid, lo), jnp.where(ok, hi, mid)

    lo, _hi = lax.fori_loop(0, 31, step, (jnp.int32(0), jnp.int32(_HI_BITS)))
    t_val = lax.bitcast_convert_type(lo, jnp.float32)
    gtm = bits > lo
    cnt_gt = jnp.sum(gtm.astype(jnp.float32))
    sum_gt = jnp.sum(jnp.where(gtm, negv, 0.0))
    sum_neg_top = sum_gt + (n_pos - cnt_gt) * t_val

    sum_neg = jnp.where(n_neg > n_pos, sum_neg_top, sum_neg_all)
    k_min = jnp.minimum(n_pos, n_neg)
    cls = (sum_pos + sum_neg) / (n_pos + k_min)
    box = box_sq / (n_box * 4.0) * 0.5
    land = land_sq / (n_land * 10.0) * 0.5
    out_ref[0, 0] = cls + box + land


@jax.jit
def _run(pred_label, pred_offset, pred_landmarks, gt_label, gt_boxes,
         gt_landmarks):
    negv, parts = _sc_stage(pred_label, pred_offset, pred_landmarks,
                            gt_label, gt_boxes, gt_landmarks)
    negv2 = negv.reshape(_R, _L)
    parts2 = parts.reshape(_NW, 128)
    return pl.pallas_call(
        _tc_body,
        in_specs=[
            pl.BlockSpec((_R, _L), lambda: (0, 0)),
            pl.BlockSpec((_NW, 128), lambda: (0, 0)),
        ],
        out_specs=pl.BlockSpec(memory_space=pltpu.SMEM),
        out_shape=jax.ShapeDtypeStruct((1, 1), jnp.float32),
    )(negv2, parts2)


def kernel(pred_label, pred_offset, pred_landmarks, gt_label, gt_boxes,
           gt_landmarks):
    out = _run(pred_label, pred_offset, pred_landmarks, gt_label, gt_boxes,
               gt_landmarks)
    return out[0, 0]


# speedup vs baseline: 1.0007x; 1.0007x over previous
import jax
import jax.numpy as jnp
from jax import lax
from jax.experimental import pallas as pl
from jax.experimental.pallas import tpu as pltpu
from jax.experimental.pallas import tpu_sc as plsc

_N = 262144
_L = 128
_R = _N // _L
_NC = 2
_NS = 16
_NW = _NC * _NS
_EPT = _N // _NW
_CH = 2048
_NCHUNK = _EPT // _CH
_HI_BITS = 0x43000000
_LN2 = 0.6931471805599453


def _softlog(x):
    bits = lax.bitcast_convert_type(x, jnp.int32)
    e = ((bits >> 23) - 127).astype(jnp.float32)
    m = lax.bitcast_convert_type(
        (bits & 0x7FFFFF) | 0x3F800000, jnp.float32)
    r = (m - 1.0) / (m + 1.0)
    r2 = r * r
    lnm = 2.0 * r * (1.0 + r2 * (1.0 / 3.0 + r2 * (0.2 + r2 * (1.0 / 7.0))))
    return e * _LN2 + lnm


def _sc_body(pl_hbm, off_hbm, plm_hbm, gl_hbm, gb_hbm, glm_hbm,
             negv_out, part_out,
             lbl_b, p_b, off_b, gb_b, plm_b, glm_b, negc_b, part_b, sem):
    wid = lax.axis_index("s") * _NC + lax.axis_index("c")
    iota = lax.iota(jnp.int32, 16)
    zf = jnp.zeros((16,), jnp.float32)

    np_a, nn_a, sp_a, sna_a, nb_a, nl_a = zf, zf, zf, zf, zf, zf
    bsq_a, lsq_a = zf, zf

    def chunk_body(c, carry):
        np_a, nn_a, sp_a, sna_a, nb_a, nl_a, bsq_a, lsq_a = carry
        s0 = wid * _EPT + c * _CH
        r1 = s0 // _L
        r4 = (s0 * 4) // _L
        r10 = (s0 * 10) // _L
        cps = [
            pltpu.make_async_copy(gl_hbm.at[pl.ds(s0, _CH)], lbl_b, sem),
            pltpu.make_async_copy(
                pl_hbm.at[pl.ds(r1, _CH // _L), :], p_b, sem),
            pltpu.make_async_copy(
                off_hbm.at[pl.ds(r4, _CH * 4 // _L), :], off_b, sem),
            pltpu.make_async_copy(
                gb_hbm.at[pl.ds(r4, _CH * 4 // _L), :], gb_b, sem),
            pltpu.make_async_copy(
                plm_hbm.at[pl.ds(r10, _CH * 10 // _L), :], plm_b, sem),
            pltpu.make_async_copy(
                glm_hbm.at[pl.ds(r10, _CH * 10 // _L), :], glm_b, sem),
        ]
        for cp in cps:
            cp.start()
        for cp in cps:
            cp.wait()

        def bce_step(g, carry):
            np_a, nn_a, sp_a, sna_a, nb_a, nl_a = carry
            li = g * 16 + iota
            lbl = lbl_b[pl.ds(g * 16, 16)]
            p = plsc.load_gather(p_b, [li >> 7, li & 127])
            lp = _softlog(p)
            l1p = _softlog(1.0 - p)
            loss_pos = -jnp.maximum(lp, -100.0)
            loss_neg = -jnp.maximum(l1p, -100.0)
            pos = lbl == 1
            neg = lbl == 0
            np_a = np_a + jnp.where(pos, 1.0, 0.0)
            nn_a = nn_a + jnp.where(neg, 1.0, 0.0)
            sp_a = sp_a + jnp.where(pos, loss_pos, 0.0)
            sna_a = sna_a + jnp.where(neg, loss_neg, 0.0)
            nb_a = nb_a + jnp.where(pos | (lbl == 2), 1.0, 0.0)
            nl_a = nl_a + jnp.where(lbl == -1, 1.0, 0.0)
            negc_b[pl.ds(g * 16, 16)] = jnp.where(neg, loss_neg, -1.0)
            return np_a, nn_a, sp_a, sna_a, nb_a, nl_a

        np_a, nn_a, sp_a, sna_a, nb_a, nl_a = lax.fori_loop(
            0, _CH // 16, bce_step,
            (np_a, nn_a, sp_a, sna_a, nb_a, nl_a), unroll=2)

        pltpu.sync_copy(negc_b, negv_out.at[pl.ds(s0, _CH)])

        def box_step(g, acc):
            q = g * 16 + iota
            a = plsc.load_gather(off_b, [q >> 7, q & 127])
            b = plsc.load_gather(gb_b, [q >> 7, q & 127])
            lbl = plsc.load_gather(lbl_b, [q >> 2])
            d = a - b
            m = (lbl == 1) | (lbl == 2)
            return acc + jnp.where(m, d * d, 0.0)

        bsq_a = lax.fori_loop(0, _CH * 4 // 16, box_step, bsq_a, unroll=4)

        def land_step(g, acc):
            q = g * 16 + iota
            a = plsc.load_gather(plm_b, [q >> 7, q & 127])
            b = plsc.load_gather(glm_b, [q >> 7, q & 127])
            lbl = plsc.load_gather(lbl_b, [q // 10])
            d = a - b
            return acc + jnp.where(lbl == -1, d * d, 0.0)

        lsq_a = lax.fori_loop(0, _CH * 10 // 16, land_step, lsq_a, unroll=4)
        return np_a, nn_a, sp_a, sna_a, nb_a, nl_a, bsq_a, lsq_a

    np_a, nn_a, sp_a, sna_a, nb_a, nl_a, bsq_a, lsq_a = lax.fori_loop(
        0, _NCHUNK, chunk_body,
        (np_a, nn_a, sp_a, sna_a, nb_a, nl_a, bsq_a, lsq_a))

    for j, vec in enumerate(
            (np_a, nn_a, sp_a, sna_a, nb_a, bsq_a, nl_a, lsq_a)):
        part_b[pl.ds(j * 16, 16)] = vec
    pltpu.sync_copy(part_b, part_out.at[pl.ds(wid * 128, 128)])


def _sc_stage(pl2, off2, plm2, gt_label, gb2, glm2):
    return pl.kernel(
        _sc_body,
        out_type=(
            jax.ShapeDtypeStruct((_N,), jnp.float32),
            jax.ShapeDtypeStruct((_NW * 128,), jnp.float32),
        ),
        mesh=plsc.VectorSubcoreMesh(core_axis_name="c", subcore_axis_name="s"),
        compiler_params=pltpu.CompilerParams(
            needs_layout_passes=False, use_tc_tiling_on_sc=False),
        scratch_types=[
            pltpu.VMEM((_CH,), jnp.int32),
            pltpu.VMEM((_CH // _L, _L), jnp.float32),
            pltpu.VMEM((_CH * 4 // _L, _L), jnp.float32),
            pltpu.VMEM((_CH * 4 // _L, _L), jnp.float32),
            pltpu.VMEM((_CH * 10 // _L, _L), jnp.float32),
            pltpu.VMEM((_CH * 10 // _L, _L), jnp.float32),
            pltpu.VMEM((_CH,), jnp.float32),
            pltpu.VMEM((128,), jnp.float32),
            pltpu.SemaphoreType.DMA,
        ],
    )(pl2, off2, plm2, gt_label, gb2, glm2)


def _tc_body(negv_ref, part_ref, out_ref):
    part = part_ref[...]
    n_pos = jnp.sum(part[:, 0:16])
    n_neg = jnp.sum(part[:, 16:32])
    sum_pos = jnp.sum(part[:, 32:48])
    sum_neg_all = jnp.sum(part[:, 48:64])
    n_box = jnp.sum(part[:, 64:80])
    box_sq = jnp.sum(part[:, 80:96])
    n_land = jnp.sum(part[:, 96:112])
    land_sq = jnp.sum(part[:, 112:128])

    negv = negv_ref[...]
    bits = lax.bitcast_convert_type(negv, jnp.int32)
    k_i = n_pos.astype(jnp.int32)

    def step(_, carry):
        lo, hi = carry
        mid = (lo + hi) // 2
        cnt = jnp.sum((bits >= mid).astype(jnp.int32))
        ok = cnt >= k_i
        return jnp.where(ok, mid, lo), jnp.where(ok, hi, mid)

    lo, _hi = lax.fori_loop(0, 31, step, (jnp.int32(0), jnp.int32(_HI_BITS)))
    t_val = lax.bitcast_convert_type(lo, jnp.float32)
    gtm = bits > lo
    cnt_gt = jnp.sum(gtm.astype(jnp.float32))
    sum_gt = jnp.sum(jnp.where(gtm, negv, 0.0))
    sum_neg_top = sum_gt + (n_pos - cnt_gt) * t_val

    sum_neg = jnp.where(n_neg > n_pos, sum_neg_top, sum_neg_all)
    k_min = jnp.minimum(n_pos, n_neg)
    cls = (sum_pos + sum_neg) / (n_pos + k_min)
    box = box_sq / (n_box * 4.0) * 0.5
    land = land_sq / (n_land * 10.0) * 0.5
    out_ref[0, 0] = cls + box + land


@jax.jit
def _run(pred_label, pred_offset, pred_landmarks, gt_label, gt_boxes,
         gt_landmarks):
    pl2 = pred_label.reshape(_R, _L)
    off2 = pred_offset.reshape(_N * 4 // _L, _L)
    gb2 = gt_boxes.reshape(_N * 4 // _L, _L)
    plm2 = pred_landmarks.reshape(_N * 10 // _L, _L)
    glm2 = gt_landmarks.reshape(_N * 10 // _L, _L)
    negv, parts = _sc_stage(pl2, off2, plm2, gt_label, gb2, glm2)
    negv2 = negv.reshape(_R, _L)
    parts2 = parts.reshape(_NW, 128)
    return pl.pallas_call(
        _tc_body,
        in_specs=[
            pl.BlockSpec((_R, _L), lambda: (0, 0)),
            pl.BlockSpec((_NW, 128), lambda: (0, 0)),
        ],
        out_specs=pl.BlockSpec(memory_space=pltpu.SMEM),
        out_shape=jax.ShapeDtypeStruct((1, 1), jnp.float32),
    )(negv2, parts2)


def kernel(pred_label, pred_offset, pred_landmarks, gt_label, gt_boxes,
           gt_landmarks):
    out = _run(pred_label, pred_offset, pred_landmarks, gt_label, gt_boxes,
               gt_landmarks)
    return out[0, 0]
